# baseline (speedup 1.0000x reference)
import functools
import math

import jax
import jax.numpy as jnp
from jax import lax
from jax.experimental import pallas as pl
from jax.experimental.pallas import tpu as pltpu
from jax.experimental.pallas import tpu_sc as plsc

_THR = math.log(0.5)
_L = 16
_CHUNK = 128


def _sc_body(t, v, ctc_f, alpha_f, mask_f, out_hbm,
             idx_v, data_v, alpha_v, mask_v, stage_v, all_v, shared, sem):
    c = lax.axis_index("c")
    s = lax.axis_index("s")

    @pl.when(c == 0)
    def _work():
        lane = lax.iota(jnp.int32, _L)
        base = s * t

        def build(i, _):
            r = base + i * _L + lane
            idx_v[pl.ds(i * _L, _L)] = (
                (r >> 3) * (8 * v) + (r & 7) * 128)
            return 0

        lax.fori_loop(0, t // _L, build, 0)

        copies = [
            pltpu.async_copy(
                ctc_f.at[idx_v.at[pl.ds(k * _CHUNK, _CHUNK)]],
                data_v.at[pl.ds(k * _CHUNK, _CHUNK)],
                sem,
            )
            for k in range(t // _CHUNK)
        ]
        pltpu.sync_copy(alpha_f.at[pl.ds(base, t)], alpha_v)
        pltpu.sync_copy(mask_f.at[pl.ds(base, t)], mask_v)
        for cp in copies:
            cp.wait()

        def comp(m, carry):
            s_acc, c_acc = carry
            a = alpha_v[pl.ds(m * _L, _L)]
            mk = mask_v[pl.ds(m * _L, _L)]
            blank = data_v[pl.ds(m * _L, _L)]
            trig = ((1.0 - blank) > _THR) & (mk != 0.0)
            return (s_acc + a, c_acc + jnp.where(trig, 1.0, 0.0))

        init = (jnp.full((_L,), 0.0, jnp.float32),
                jnp.full((_L,), 0.0, jnp.float32))
        s_acc, c_acc = lax.fori_loop(0, t // _L, comp, init)

        row_sum = jnp.sum(s_acc)
        cnt = jnp.sum(c_acc)
        term = jnp.abs(row_sum - 1.0) * cnt
        inc = jnp.where(cnt > 0.5, 1.0, 0.0)
        stage_v[...] = (jnp.where(lane == 0, term, 0.0)
                        + jnp.where(lane == 1, inc, 0.0))
        pltpu.sync_copy(stage_v, shared.at[pl.ds(s * _L, _L)])

    plsc.subcore_barrier()

    @pl.when(jnp.logical_and(c == 0, s == 0))
    def _combine():
        pltpu.sync_copy(shared, all_v)
        lane = lax.iota(jnp.int32, _L)
        acc = jnp.full((_L,), 0.0, jnp.float32)
        for r in range(_L):
            acc = acc + all_v[pl.ds(r * _L, _L)]
        num = jnp.sum(jnp.where(lane == 0, acc, 0.0))
        den = jnp.sum(jnp.where(lane == 1, acc, 0.0))
        num_v = jnp.full((_L,), 0.0, jnp.float32) + num
        den_v = jnp.full((_L,), 0.0, jnp.float32) + den
        stage_v[...] = jnp.where(den_v > 0.0, num_v / den_v, 0.0)
        pltpu.sync_copy(stage_v, out_hbm)


def kernel(alpha, ctc_log_probs, mask):
    b, t = alpha.shape
    v = ctc_log_probs.shape[-1]
    rows = b * t
    ctc_f = (ctc_log_probs.reshape(rows // 8, 8, v // 128, 128)
             .transpose(0, 2, 1, 3).reshape(-1))
    alpha_f = alpha.reshape(-1)
    mask_f = mask.reshape(-1)

    mesh = plsc.VectorSubcoreMesh(core_axis_name="c", subcore_axis_name="s")
    run = pl.kernel(
        functools.partial(_sc_body, t, v),
        mesh=mesh,
        compiler_params=pltpu.CompilerParams(needs_layout_passes=False),
        out_type=jax.ShapeDtypeStruct((_L,), jnp.float32),
        scratch_types=[
            pltpu.VMEM((t,), jnp.int32),
            pltpu.VMEM((t,), jnp.float32),
            pltpu.VMEM((t,), jnp.float32),
            pltpu.VMEM((t,), jnp.float32),
            pltpu.VMEM((_L,), jnp.float32),
            pltpu.VMEM((_L * _L,), jnp.float32),
            pltpu.VMEM_SHARED((_L * _L,), jnp.float32),
            pltpu.SemaphoreType.DMA,
        ],
    )
    out = run(ctc_f, alpha_f, mask_f)
    return out[0]


# speedup vs baseline: 1.0649x; 1.0649x over previous
import functools
import math

import jax
import jax.numpy as jnp
from jax import lax
from jax.experimental import pallas as pl
from jax.experimental.pallas import tpu as pltpu
from jax.experimental.pallas import tpu_sc as plsc

_THR = math.log(0.5)
_L = 16
_CHUNK = 128


def _sc_body(t, v, ctc_f, alpha_f, mask_f, out_hbm,
             idx_v, data_v, alpha_v, mask_v, stage_v, all_v, shared, sem):
    c = lax.axis_index("c")
    s = lax.axis_index("s")

    @pl.when(c == 0)
    def _work():
        lane = lax.iota(jnp.int32, _L)
        base = s * t

        def build(i, _):
            r = base + i * _L + lane
            idx_v[pl.ds(i * _L, _L)] = (
                (r >> 3) * (8 * v) + (r & 7) * 128)
            return 0

        lax.fori_loop(0, t // _L, build, 0)

        copies = [
            pltpu.async_copy(
                ctc_f.at[idx_v.at[pl.ds(k * _CHUNK, _CHUNK)]],
                data_v.at[pl.ds(k * _CHUNK, _CHUNK)],
                sem,
            )
            for k in range(t // _CHUNK)
        ]
        pltpu.sync_copy(alpha_f.at[pl.ds(base, t)], alpha_v)
        pltpu.sync_copy(mask_f.at[pl.ds(base, t)], mask_v)
        for cp in copies:
            cp.wait()

        def comp(m, carry):
            s_acc, c_acc = carry
            a = alpha_v[pl.ds(m * _L, _L)]
            mk = mask_v[pl.ds(m * _L, _L)]
            blank = data_v[pl.ds(m * _L, _L)]
            trig = ((1.0 - blank) > _THR) & (mk != 0.0)
            return (s_acc + a, c_acc + jnp.where(trig, 1.0, 0.0))

        init = (jnp.full((_L,), 0.0, jnp.float32),
                jnp.full((_L,), 0.0, jnp.float32))
        s_acc, c_acc = lax.fori_loop(0, t // _L, comp, init)

        row_sum = jnp.sum(s_acc)
        cnt = jnp.sum(c_acc)
        term = jnp.abs(row_sum - 1.0) * cnt
        inc = jnp.where(cnt > 0.5, 1.0, 0.0)
        stage_v[...] = (jnp.where(lane == 0, term, 0.0)
                        + jnp.where(lane == 1, inc, 0.0))
        pltpu.sync_copy(stage_v, shared.at[pl.ds(s * _L, _L)])

    plsc.subcore_barrier()

    @pl.when(jnp.logical_and(c == 0, s == 0))
    def _combine():
        pltpu.sync_copy(shared, all_v)
        lane = lax.iota(jnp.int32, _L)
        acc = jnp.full((_L,), 0.0, jnp.float32)
        for r in range(_L):
            acc = acc + all_v[pl.ds(r * _L, _L)]
        num = jnp.sum(jnp.where(lane == 0, acc, 0.0))
        den = jnp.sum(jnp.where(lane == 1, acc, 0.0))
        num_v = jnp.full((_L,), 0.0, jnp.float32) + num
        den_v = jnp.full((_L,), 0.0, jnp.float32) + den
        stage_v[...] = jnp.where(den_v > 0.0, num_v / den_v, 0.0)
        pltpu.sync_copy(stage_v, out_hbm)


def kernel(alpha, ctc_log_probs, mask):
    b, t = alpha.shape
    v = ctc_log_probs.shape[-1]
    rows = b * t
    ctc_f = (ctc_log_probs.reshape(rows // 8, 8, v // 128, 128)
             .transpose(0, 2, 1, 3).reshape(-1))
    alpha_f = alpha.reshape(-1)
    mask_f = mask.reshape(-1)

    mesh = plsc.VectorSubcoreMesh(core_axis_name="c", subcore_axis_name="s", num_cores=1)
    run = pl.kernel(
        functools.partial(_sc_body, t, v),
        mesh=mesh,
        compiler_params=pltpu.CompilerParams(needs_layout_passes=False),
        out_type=jax.ShapeDtypeStruct((_L,), jnp.float32),
        scratch_types=[
            pltpu.VMEM((t,), jnp.int32),
            pltpu.VMEM((t,), jnp.float32),
            pltpu.VMEM((t,), jnp.float32),
            pltpu.VMEM((t,), jnp.float32),
            pltpu.VMEM((_L,), jnp.float32),
            pltpu.VMEM((_L * _L,), jnp.float32),
            pltpu.VMEM_SHARED((_L * _L,), jnp.float32),
            pltpu.SemaphoreType.DMA,
        ],
    )
    out = run(ctc_f, alpha_f, mask_f)
    return out[0]
